# baseline (speedup 1.0000x reference)
import functools

import jax
import jax.numpy as jnp
from jax import lax
from jax.experimental import pallas as pl
from jax.experimental.pallas import tpu as pltpu
from jax.experimental.pallas import tpu_sc as plsc

NC = 2
NS = 16
L = 16
NW = NC * NS


def _sc_kernel(R, C, CH):
    RPW = R // NW
    NCH = RPW // CH
    mesh = plsc.VectorSubcoreMesh(core_axis_name="c", subcore_axis_name="s")

    @functools.partial(
        pl.kernel,
        mesh=mesh,
        compiler_params=pltpu.CompilerParams(needs_layout_passes=False),
        out_type=[
            jax.ShapeDtypeStruct((R,), jnp.int32),
            jax.ShapeDtypeStruct((R * 4,), jnp.float32),
            jax.ShapeDtypeStruct((NW * L,), jnp.int32),
        ],
        scratch_types=[
            pltpu.VMEM((CH * C,), jnp.float32),
            pltpu.VMEM((CH * 4,), jnp.float32),
            pltpu.VMEM((CH,), jnp.int32),
            pltpu.VMEM((CH * 4,), jnp.float32),
            pltpu.VMEM((L,), jnp.int32),
        ],
    )
    def kern(lg_hbm, bx_hbm, lab_hbm, bxo_hbm, cnt_hbm,
             lgb, bxb, labb, bxob, cntb):
        wid = lax.axis_index("s") * NC + lax.axis_index("c")
        base_row = wid * RPW
        lane_c = lax.iota(jnp.int32, L) * C
        lane_4 = lax.iota(jnp.int32, L) * 4
        ones = jnp.ones((L,), jnp.int32)
        zeros = jnp.zeros((L,), jnp.int32)
        neg1 = jnp.full((L,), -1, jnp.int32)

        def group_body(g, cnt):
            base = lane_c + g * (L * C)
            ms = []
            js = []
            idxs = []
            for k in range(8):
                idx = base + (32 * k)
                ms.append(plsc.load_gather(lgb, [idx]))
                js.append(zeros)
                idxs.append(idx)
            jvec = zeros
            for _ in range(31):
                jvec = jvec + 1
                for k in range(8):
                    idxs[k] = idxs[k] + 1
                    v = plsc.load_gather(lgb, [idxs[k]])
                    gt = v > ms[k]
                    ms[k] = jnp.where(gt, v, ms[k])
                    js[k] = jnp.where(gt, jvec, js[k])
            am = ms[0]
            aj = js[0]
            for k in range(1, 8):
                cj = js[k] + (32 * k)
                gt = ms[k] > am
                am = jnp.where(gt, ms[k], am)
                aj = jnp.where(gt, cj, aj)
            valid = am > 0.0
            labb[pl.ds(g * L, L)] = jnp.where(valid, aj, neg1)
            cnt = cnt + jnp.where(valid, ones, zeros)
            bbase = lane_4 + g * (L * 4)
            for k4 in range(4):
                bidx = bbase + k4
                bv = plsc.load_gather(bxb, [bidx])
                plsc.store_scatter(bxob, [bidx], jnp.where(valid, bv, 0.0))
            return cnt

        def chunk_body(ch, cnt):
            off = base_row + ch * CH
            pltpu.sync_copy(lg_hbm.at[pl.ds(off * C, CH * C)], lgb)
            pltpu.sync_copy(bx_hbm.at[pl.ds(off * 4, CH * 4)], bxb)
            cnt = lax.fori_loop(0, CH // L, group_body, cnt)
            pltpu.sync_copy(labb, lab_hbm.at[pl.ds(off, CH)])
            pltpu.sync_copy(bxob, bxo_hbm.at[pl.ds(off * 4, CH * 4)])
            return cnt

        cnt = lax.fori_loop(0, NCH, chunk_body, zeros)
        cntb[...] = cnt
        pltpu.sync_copy(cntb, cnt_hbm.at[pl.ds(wid * L, L)])

    return kern


def kernel(pred_logits, pred_boxes):
    B, Q, C = pred_logits.shape
    R = B * Q
    lgf = pred_logits.reshape(R * C)
    bxf = pred_boxes.reshape(R * 4)
    labels, boxes, cntp = _sc_kernel(R, C, 128)(lgf, bxf)
    num_boxes = jnp.maximum(jnp.sum(cntp).astype(jnp.float32), 1.0)
    return labels.reshape(B, Q), boxes.reshape(B, Q, 4), num_boxes


# speedup vs baseline: 8.7196x; 8.7196x over previous
import jax
import jax.numpy as jnp
from jax.experimental import pallas as pl
from jax.experimental.pallas import tpu as pltpu


def _body(lg_ref, bxt_ref, lab_ref, boxt_ref):
    x = lg_ref[...]
    br, c = x.shape
    q = lab_ref.shape[1]
    m = jnp.max(x, axis=-1, keepdims=True)
    eq = x == m
    cidx = jax.lax.broadcasted_iota(jnp.int32, x.shape, 1)
    sel = eq & (x > 0.0)
    a = jnp.min(jnp.where(sel, cidx, c), axis=-1, keepdims=True)
    lab_col = jnp.where(a < c, a, -1)
    rows = []
    for k in range(br // 128):
        bc = jnp.broadcast_to(lab_col[k * 128:(k + 1) * 128, :], (128, 128))
        rows.append(bc.T[0:1, :])
    lab_lane = jnp.concatenate(rows, axis=0)
    lab_ref[...] = lab_lane
    boxt_ref[...] = jnp.where((lab_lane >= 0)[None], bxt_ref[...], 0.0)


def kernel(pred_logits, pred_boxes):
    B, Q, C = pred_logits.shape
    R = B * Q
    lg = pred_logits.reshape(R, C)
    bxt = pred_boxes.reshape(R, 4).T.reshape(4, R // 128, 128)

    BR = 4096
    BL = BR // 128
    labels, boxest = pl.pallas_call(
        _body,
        grid=(R // BR,),
        in_specs=[
            pl.BlockSpec((BR, C), lambda i: (i, 0)),
            pl.BlockSpec((4, BL, 128), lambda i: (0, i, 0)),
        ],
        out_specs=[
            pl.BlockSpec((BL, 128), lambda i: (i, 0)),
            pl.BlockSpec((4, BL, 128), lambda i: (0, i, 0)),
        ],
        out_shape=[
            jax.ShapeDtypeStruct((R // 128, 128), jnp.int32),
            jax.ShapeDtypeStruct((4, R // 128, 128), jnp.float32),
        ],
    )(lg, bxt)
    labels2 = labels.reshape(R)
    num_boxes = jnp.maximum(jnp.sum(labels2 >= 0).astype(jnp.float32), 1.0)
    boxes = boxest.reshape(4, R).T.reshape(B, Q, 4)
    return labels2.reshape(B, Q), boxes, num_boxes
